# baseline (speedup 1.0000x reference)
import functools

import jax
import jax.numpy as jnp
from jax import lax
from jax.experimental import pallas as pl
from jax.experimental.pallas import tpu as pltpu
from jax.experimental.pallas import tpu_sc as plsc

TEXT_DIM = 768
HIDDEN = 256
B, L = 1024, 50
TOKENS = B * L
NC, NS = 2, 16
NW = NC * NS
PER_W = TOKENS // NW
CHUNK = 80
NCHUNK = PER_W // CHUNK

BATCH_TILE = 8
TILE_T = BATCH_TILE * L
EPS = 1e-12


def _sc_gather_body(ids_hbm, table_hbm, out_hbm, idx_v, rows_v, sem):
    wid = lax.axis_index("s") * NC + lax.axis_index("c")
    pltpu.sync_copy(ids_hbm.at[wid], idx_v)
    for j in range(NCHUNK):
        pltpu.async_copy(table_hbm.at[idx_v.at[j]], rows_v, sem).wait()
        pltpu.sync_copy(rows_v, out_hbm.at[pl.ds(wid * PER_W + j * CHUNK, CHUNK)])


@functools.cache
def _sc_gather():
    return pl.kernel(
        _sc_gather_body,
        mesh=plsc.VectorSubcoreMesh(core_axis_name="c", subcore_axis_name="s"),
        out_type=jax.ShapeDtypeStruct((TOKENS, TEXT_DIM), jnp.float32),
        scratch_types=[
            pltpu.VMEM((NCHUNK, CHUNK), jnp.int32),
            pltpu.VMEM((CHUNK, TEXT_DIM), jnp.float32),
            pltpu.SemaphoreType.DMA,
        ],
    )


def _tc_body(x_ref, t_ref, w_ref, wt_ref, g_ref, b_ref, o_ref):
    x = x_ref[...].astype(jnp.bfloat16)
    t = t_ref[...]
    y = jnp.dot(x, w_ref[...].astype(jnp.bfloat16),
                preferred_element_type=jnp.float32)
    y = y + jnp.dot(t, wt_ref[...], preferred_element_type=jnp.float32)
    mu = jnp.mean(y, axis=-1, keepdims=True)
    d = y - mu
    var = jnp.mean(d * d, axis=-1, keepdims=True)
    z = d * lax.rsqrt(var + EPS) * g_ref[...] + b_ref[...]
    for b in range(BATCH_TILE):
        o_ref[b] = z[b * L:(b + 1) * L, :]


_tc_call = pl.pallas_call(
    _tc_body,
    grid=(B // BATCH_TILE,),
    in_specs=[
        pl.BlockSpec((TILE_T, TEXT_DIM), lambda i: (i, 0)),
        pl.BlockSpec((TILE_T, 16), lambda i: (i, 0)),
        pl.BlockSpec((TEXT_DIM, HIDDEN), lambda i: (0, 0)),
        pl.BlockSpec((16, HIDDEN), lambda i: (0, 0)),
        pl.BlockSpec((1, HIDDEN), lambda i: (0, 0)),
        pl.BlockSpec((1, HIDDEN), lambda i: (0, 0)),
    ],
    out_specs=pl.BlockSpec((BATCH_TILE, L, HIDDEN), lambda i: (i, 0, 0)),
    out_shape=jax.ShapeDtypeStruct((B, L, HIDDEN), jnp.float32),
)


def kernel(input_ids, rel_ids, token_type_ids, token_table, W_proj, b_proj,
           rel_table, type_table, ln_gamma, ln_beta):
    ids = input_ids.reshape(-1).astype(jnp.int32)

    gathered = _sc_gather()(ids.reshape(NW, NCHUNK, CHUNK), token_table)

    lanes = jnp.arange(16, dtype=jnp.int32)
    tails = ((lanes[None, :] == rel_ids.reshape(-1, 1)).astype(jnp.float32)
             + (lanes[None, :] == token_type_ids.reshape(-1, 1) + 3).astype(jnp.float32)
             + (lanes[None, :] == 5).astype(jnp.float32))
    w_tail = jnp.concatenate(
        [rel_table, type_table, b_proj[None, :],
         jnp.zeros((16 - 3 - 2 - 1, HIDDEN), jnp.float32)], axis=0)

    return _tc_call(gathered, tails, W_proj, w_tail,
                    ln_gamma[None, :], ln_beta[None, :])


# speedup vs baseline: 1.1504x; 1.1504x over previous
import functools

import jax
import jax.numpy as jnp
from jax import lax
from jax.experimental import pallas as pl
from jax.experimental.pallas import tpu as pltpu
from jax.experimental.pallas import tpu_sc as plsc

TEXT_DIM = 768
HIDDEN = 256
B, L = 1024, 50
TOKENS = B * L
NC, NS = 2, 16
NW = NC * NS
PER_W = TOKENS // NW
CHUNK = 80
NCHUNK = PER_W // CHUNK

BATCH_TILE = 16
TILE_T = BATCH_TILE * L
EPS = 1e-12


def _sc_gather_body(ids_hbm, table_hbm, out_hbm, idx_v, rows_v, sem):
    wid = lax.axis_index("s") * NC + lax.axis_index("c")
    pltpu.sync_copy(ids_hbm.at[wid], idx_v)
    for j in range(NCHUNK):
        pltpu.async_copy(table_hbm.at[idx_v.at[j]], rows_v, sem).wait()
        pltpu.sync_copy(rows_v, out_hbm.at[pl.ds(wid * PER_W + j * CHUNK, CHUNK)])


@functools.cache
def _sc_gather():
    return pl.kernel(
        _sc_gather_body,
        mesh=plsc.VectorSubcoreMesh(core_axis_name="c", subcore_axis_name="s"),
        out_type=jax.ShapeDtypeStruct((TOKENS, TEXT_DIM), jnp.float32),
        scratch_types=[
            pltpu.VMEM((NCHUNK, CHUNK), jnp.int32),
            pltpu.VMEM((CHUNK, TEXT_DIM), jnp.float32),
            pltpu.SemaphoreType.DMA,
        ],
    )


def _tc_body(x_ref, t_ref, w_ref, wt_ref, g_ref, b_ref, o_ref):
    x = x_ref[...].astype(jnp.bfloat16)
    t = t_ref[...]
    y = jnp.dot(x, w_ref[...].astype(jnp.bfloat16),
                preferred_element_type=jnp.float32)
    y = y + jnp.dot(t, wt_ref[...], preferred_element_type=jnp.float32)
    mu = jnp.mean(y, axis=-1, keepdims=True)
    d = y - mu
    var = jnp.mean(d * d, axis=-1, keepdims=True)
    z = d * lax.rsqrt(var + EPS) * g_ref[...] + b_ref[...]
    for b in range(BATCH_TILE):
        o_ref[b] = z[b * L:(b + 1) * L, :]


_tc_call = pl.pallas_call(
    _tc_body,
    grid=(B // BATCH_TILE,),
    in_specs=[
        pl.BlockSpec((TILE_T, TEXT_DIM), lambda i: (i, 0)),
        pl.BlockSpec((TILE_T, 16), lambda i: (i, 0)),
        pl.BlockSpec((TEXT_DIM, HIDDEN), lambda i: (0, 0)),
        pl.BlockSpec((16, HIDDEN), lambda i: (0, 0)),
        pl.BlockSpec((1, HIDDEN), lambda i: (0, 0)),
        pl.BlockSpec((1, HIDDEN), lambda i: (0, 0)),
    ],
    out_specs=pl.BlockSpec((BATCH_TILE, L, HIDDEN), lambda i: (i, 0, 0)),
    out_shape=jax.ShapeDtypeStruct((B, L, HIDDEN), jnp.float32),
)


def kernel(input_ids, rel_ids, token_type_ids, token_table, W_proj, b_proj,
           rel_table, type_table, ln_gamma, ln_beta):
    ids = input_ids.reshape(-1).astype(jnp.int32)

    gathered = _sc_gather()(ids.reshape(NW, NCHUNK, CHUNK), token_table)

    lanes = jnp.arange(16, dtype=jnp.int32)
    tails = ((lanes[None, :] == rel_ids.reshape(-1, 1)).astype(jnp.float32)
             + (lanes[None, :] == token_type_ids.reshape(-1, 1) + 3).astype(jnp.float32)
             + (lanes[None, :] == 5).astype(jnp.float32))
    w_tail = jnp.concatenate(
        [rel_table, type_table, b_proj[None, :],
         jnp.zeros((16 - 3 - 2 - 1, HIDDEN), jnp.float32)], axis=0)

    return _tc_call(gathered, tails, W_proj, w_tail,
                    ln_gamma[None, :], ln_beta[None, :])


# speedup vs baseline: 1.2382x; 1.0763x over previous
import functools

import jax
import jax.numpy as jnp
from jax import lax
from jax.experimental import pallas as pl
from jax.experimental.pallas import tpu as pltpu
from jax.experimental.pallas import tpu_sc as plsc

TEXT_DIM = 768
HIDDEN = 256
B, L = 1024, 50
TOKENS = B * L
NC, NS = 2, 16
NW = NC * NS
PER_W = TOKENS // NW
CHUNK = 80
NCHUNK = PER_W // CHUNK

BATCH_TILE = 32
TILE_T = BATCH_TILE * L
EPS = 1e-12


def _sc_gather_body(ids_hbm, table_hbm, out_hbm, idx_v, rows_v, sem):
    wid = lax.axis_index("s") * NC + lax.axis_index("c")
    pltpu.sync_copy(ids_hbm.at[wid], idx_v)
    for j in range(NCHUNK):
        pltpu.async_copy(table_hbm.at[idx_v.at[j]], rows_v, sem).wait()
        pltpu.sync_copy(rows_v, out_hbm.at[pl.ds(wid * PER_W + j * CHUNK, CHUNK)])


@functools.cache
def _sc_gather():
    return pl.kernel(
        _sc_gather_body,
        mesh=plsc.VectorSubcoreMesh(core_axis_name="c", subcore_axis_name="s"),
        out_type=jax.ShapeDtypeStruct((TOKENS, TEXT_DIM), jnp.float32),
        scratch_types=[
            pltpu.VMEM((NCHUNK, CHUNK), jnp.int32),
            pltpu.VMEM((CHUNK, TEXT_DIM), jnp.float32),
            pltpu.SemaphoreType.DMA,
        ],
    )


def _tc_body(x_ref, t_ref, w_ref, wt_ref, g_ref, b_ref, o_ref):
    x = x_ref[...].astype(jnp.bfloat16)
    t = t_ref[...]
    y = jnp.dot(x, w_ref[...].astype(jnp.bfloat16),
                preferred_element_type=jnp.float32)
    y = y + jnp.dot(t, wt_ref[...], preferred_element_type=jnp.float32)
    mu = jnp.mean(y, axis=-1, keepdims=True)
    d = y - mu
    var = jnp.mean(d * d, axis=-1, keepdims=True)
    z = d * lax.rsqrt(var + EPS) * g_ref[...] + b_ref[...]
    for b in range(BATCH_TILE):
        o_ref[b] = z[b * L:(b + 1) * L, :]


_tc_call = pl.pallas_call(
    _tc_body,
    grid=(B // BATCH_TILE,),
    in_specs=[
        pl.BlockSpec((TILE_T, TEXT_DIM), lambda i: (i, 0)),
        pl.BlockSpec((TILE_T, 16), lambda i: (i, 0)),
        pl.BlockSpec((TEXT_DIM, HIDDEN), lambda i: (0, 0)),
        pl.BlockSpec((16, HIDDEN), lambda i: (0, 0)),
        pl.BlockSpec((1, HIDDEN), lambda i: (0, 0)),
        pl.BlockSpec((1, HIDDEN), lambda i: (0, 0)),
    ],
    out_specs=pl.BlockSpec((BATCH_TILE, L, HIDDEN), lambda i: (i, 0, 0)),
    out_shape=jax.ShapeDtypeStruct((B, L, HIDDEN), jnp.float32),
)


def kernel(input_ids, rel_ids, token_type_ids, token_table, W_proj, b_proj,
           rel_table, type_table, ln_gamma, ln_beta):
    ids = input_ids.reshape(-1).astype(jnp.int32)

    gathered = _sc_gather()(ids.reshape(NW, NCHUNK, CHUNK), token_table)

    lanes = jnp.arange(16, dtype=jnp.int32)
    tails = ((lanes[None, :] == rel_ids.reshape(-1, 1)).astype(jnp.float32)
             + (lanes[None, :] == token_type_ids.reshape(-1, 1) + 3).astype(jnp.float32)
             + (lanes[None, :] == 5).astype(jnp.float32))
    w_tail = jnp.concatenate(
        [rel_table, type_table, b_proj[None, :],
         jnp.zeros((16 - 3 - 2 - 1, HIDDEN), jnp.float32)], axis=0)

    return _tc_call(gathered, tails, W_proj, w_tail,
                    ln_gamma[None, :], ln_beta[None, :])


# speedup vs baseline: 1.2562x; 1.0146x over previous
import functools

import jax
import jax.numpy as jnp
from jax import lax
from jax.experimental import pallas as pl
from jax.experimental.pallas import tpu as pltpu
from jax.experimental.pallas import tpu_sc as plsc

TEXT_DIM = 768
HIDDEN = 256
B, L = 1024, 50
TOKENS = B * L
NC, NS = 2, 16
NW = NC * NS
PER_W = TOKENS // NW
CHUNK = 80
NCHUNK = PER_W // CHUNK

BATCH_TILE = 64
TILE_T = BATCH_TILE * L
EPS = 1e-12


def _sc_gather_body(ids_hbm, table_hbm, out_hbm, idx_v, rows_v, sem):
    wid = lax.axis_index("s") * NC + lax.axis_index("c")
    pltpu.sync_copy(ids_hbm.at[wid], idx_v)
    for j in range(NCHUNK):
        pltpu.async_copy(table_hbm.at[idx_v.at[j]], rows_v, sem).wait()
        pltpu.sync_copy(rows_v, out_hbm.at[pl.ds(wid * PER_W + j * CHUNK, CHUNK)])


@functools.cache
def _sc_gather():
    return pl.kernel(
        _sc_gather_body,
        mesh=plsc.VectorSubcoreMesh(core_axis_name="c", subcore_axis_name="s"),
        out_type=jax.ShapeDtypeStruct((TOKENS, TEXT_DIM), jnp.float32),
        scratch_types=[
            pltpu.VMEM((NCHUNK, CHUNK), jnp.int32),
            pltpu.VMEM((CHUNK, TEXT_DIM), jnp.float32),
            pltpu.SemaphoreType.DMA,
        ],
    )


def _tc_body(x_ref, t_ref, w_ref, wt_ref, g_ref, b_ref, o_ref):
    x = x_ref[...].astype(jnp.bfloat16)
    t = t_ref[...]
    y = jnp.dot(x, w_ref[...].astype(jnp.bfloat16),
                preferred_element_type=jnp.float32)
    y = y + jnp.dot(t, wt_ref[...], preferred_element_type=jnp.float32)
    mu = jnp.mean(y, axis=-1, keepdims=True)
    d = y - mu
    var = jnp.mean(d * d, axis=-1, keepdims=True)
    z = d * lax.rsqrt(var + EPS) * g_ref[...] + b_ref[...]
    for b in range(BATCH_TILE):
        o_ref[b] = z[b * L:(b + 1) * L, :]


_tc_call = pl.pallas_call(
    _tc_body,
    grid=(B // BATCH_TILE,),
    in_specs=[
        pl.BlockSpec((TILE_T, TEXT_DIM), lambda i: (i, 0)),
        pl.BlockSpec((TILE_T, 16), lambda i: (i, 0)),
        pl.BlockSpec((TEXT_DIM, HIDDEN), lambda i: (0, 0)),
        pl.BlockSpec((16, HIDDEN), lambda i: (0, 0)),
        pl.BlockSpec((1, HIDDEN), lambda i: (0, 0)),
        pl.BlockSpec((1, HIDDEN), lambda i: (0, 0)),
    ],
    out_specs=pl.BlockSpec((BATCH_TILE, L, HIDDEN), lambda i: (i, 0, 0)),
    out_shape=jax.ShapeDtypeStruct((B, L, HIDDEN), jnp.float32),
)


def kernel(input_ids, rel_ids, token_type_ids, token_table, W_proj, b_proj,
           rel_table, type_table, ln_gamma, ln_beta):
    ids = input_ids.reshape(-1).astype(jnp.int32)

    gathered = _sc_gather()(ids.reshape(NW, NCHUNK, CHUNK), token_table)

    lanes = jnp.arange(16, dtype=jnp.int32)
    tails = ((lanes[None, :] == rel_ids.reshape(-1, 1)).astype(jnp.float32)
             + (lanes[None, :] == token_type_ids.reshape(-1, 1) + 3).astype(jnp.float32)
             + (lanes[None, :] == 5).astype(jnp.float32))
    w_tail = jnp.concatenate(
        [rel_table, type_table, b_proj[None, :],
         jnp.zeros((16 - 3 - 2 - 1, HIDDEN), jnp.float32)], axis=0)

    return _tc_call(gathered, tails, W_proj, w_tail,
                    ln_gamma[None, :], ln_beta[None, :])
